# Initial kernel scaffold; baseline (speedup 1.0000x reference)
import functools

import jax
import jax.numpy as jnp
from jax import lax
from jax.experimental import pallas as pl
from jax.experimental.pallas import tpu as pltpu
from jax.experimental.pallas import tpu_sc as plsc

VOCAB = 1_000_000
D = 64
B_TOTAL = 16384 * 50

NC, NS = 2, 16
NW = NC * NS

G = 128
K = 4
S = K * G
NBUF = 2

IDX_ROWS = B_TOTAL // G
ROWS_PER_W = IDX_ROWS // NW
NSTEPS = ROWS_PER_W // K

_mesh = plsc.VectorSubcoreMesh(
    core_axis_name="c", subcore_axis_name="s", num_cores=NC, num_subcores=NS
)


@functools.partial(
    pl.kernel,
    out_type=jax.ShapeDtypeStruct((B_TOTAL, D), jnp.float32),
    mesh=_mesh,
    scratch_types=[
        pltpu.VMEM((NBUF, K, G), jnp.int32),
        pltpu.VMEM((NBUF, S, D), jnp.float32),
        pltpu.SemaphoreType.DMA,
        pltpu.SemaphoreType.DMA,
    ],
)
def _embed_gather(table_hbm, idx_hbm, out_hbm, idx_v, rows_v, sem0, sem1):
    sems = (sem0, sem1)
    wid = lax.axis_index("s") * NC + lax.axis_index("c")
    row0 = wid * ROWS_PER_W

    def fire(slot, s):
        pltpu.sync_copy(idx_hbm.at[pl.ds(row0 + s * K, K)], idx_v.at[slot])
        for j in range(K):
            pltpu.async_copy(
                table_hbm.at[idx_v.at[slot, j]],
                rows_v.at[slot, pl.ds(j * G, G)],
                sems[slot],
            )

    def drain_flush(slot, s):
        pltpu.make_async_copy(
            out_hbm.at[pl.ds(0, S)], rows_v.at[slot], sems[slot]
        ).wait()
        pltpu.sync_copy(
            rows_v.at[slot], out_hbm.at[pl.ds((row0 + s * K) * G, S)]
        )

    for b in range(NBUF):
        fire(b, b)

    @pl.loop(0, NSTEPS, step=NBUF)
    def _(g):
        for b in range(NBUF):
            s = g + b
            drain_flush(b, s)

            @pl.when(s + NBUF < NSTEPS)
            def _():
                fire(b, s + NBUF)


def kernel(token_ids, embeddings):
    idx2d = token_ids.reshape(-1).astype(jnp.int32).reshape(IDX_ROWS, G)
    out = _embed_gather(embeddings, idx2d)
    return out.reshape(token_ids.shape + (D,))


# speedup vs baseline: 1.8558x; 1.8558x over previous
import functools

import jax
import jax.numpy as jnp
from jax import lax
from jax.experimental import pallas as pl
from jax.experimental.pallas import tpu as pltpu
from jax.experimental.pallas import tpu_sc as plsc

VOCAB = 1_000_000
D = 64
B_TOTAL = 16384 * 50

NC, NS = 2, 16
NW = NC * NS

G = 128
K = 4
S = K * G
NBUF = 2

IDX_ROWS = B_TOTAL // G
ROWS_PER_W = IDX_ROWS // NW
NSTEPS = ROWS_PER_W // K

_mesh = plsc.VectorSubcoreMesh(
    core_axis_name="c", subcore_axis_name="s", num_cores=NC, num_subcores=NS
)


@functools.partial(
    pl.kernel,
    out_type=jax.ShapeDtypeStruct((B_TOTAL, D), jnp.float32),
    mesh=_mesh,
    scratch_types=[
        pltpu.VMEM((NBUF, K, G), jnp.int32),
        pltpu.VMEM((NBUF, S, D), jnp.float32),
        pltpu.SemaphoreType.DMA,
        pltpu.SemaphoreType.DMA,
    ],
    compiler_params=pltpu.CompilerParams(use_tc_tiling_on_sc=False),
)
def _embed_gather(table_hbm, idx_hbm, out_hbm, idx_v, rows_v, sem0, sem1):
    sems = (sem0, sem1)
    wid = lax.axis_index("s") * NC + lax.axis_index("c")
    row0 = wid * ROWS_PER_W

    def fire(slot, s):
        pltpu.sync_copy(idx_hbm.at[pl.ds(row0 + s * K, K)], idx_v.at[slot])
        for j in range(K):
            pltpu.async_copy(
                table_hbm.at[idx_v.at[slot, j]],
                rows_v.at[slot, pl.ds(j * G, G)],
                sems[slot],
            )

    def drain_flush(slot, s):
        pltpu.make_async_copy(
            out_hbm.at[pl.ds(0, S)], rows_v.at[slot], sems[slot]
        ).wait()
        pltpu.sync_copy(
            rows_v.at[slot], out_hbm.at[pl.ds((row0 + s * K) * G, S)]
        )

    for b in range(NBUF):
        fire(b, b)

    @pl.loop(0, NSTEPS, step=NBUF)
    def _(g):
        for b in range(NBUF):
            s = g + b
            drain_flush(b, s)

            @pl.when(s + NBUF < NSTEPS)
            def _():
                fire(b, s + NBUF)


def kernel(token_ids, embeddings):
    idx2d = token_ids.reshape(-1).astype(jnp.int32).reshape(IDX_ROWS, G)
    out = _embed_gather(embeddings, idx2d)
    return out.reshape(token_ids.shape + (D,))
